# baseline (speedup 1.0000x reference)
import jax
import jax.numpy as jnp
from jax import lax
from jax.experimental import pallas as pl
from jax.experimental.pallas import tpu as pltpu
from jax.experimental.pallas import tpu_sc as plsc

_B = 4096
_N = 4096
_K = 2048
_L = 16
_NW = 32
_N_SC = 512
_SC_ROUNDS = _N_SC // (_NW * _L)
_W = 256
_N_TC = _N - _N_SC


def _sc_body(x_hbm, o_hbm, chunk, hist):
    wid = lax.axis_index("c") * 16 + lax.axis_index("s")
    lanes = lax.iota(jnp.int32, 16)
    ones = jnp.ones((_L,), jnp.int32)
    zeros = jnp.zeros((_L,), jnp.int32)
    sign = jnp.full((_L,), -0x80000000, jnp.int32)
    m7f = jnp.full((_L,), 0x7FFFFFFF, jnp.int32)

    @plsc.parallel_loop(0, 256, unroll=8)
    def _(b):
        hist[b, :] = zeros

    def scan_hist(krem):
        def sbody(j, car):
            s, bsel, above, crossed = car
            b = 255 - j
            hv = hist[b, :]
            hist[b, :] = zeros
            s2 = s + hv
            cross_now = (s2 >= krem) & (crossed == 0)
            bsel = jnp.where(cross_now, b, bsel)
            above = jnp.where(cross_now, s, above)
            crossed = jnp.where(cross_now, ones, crossed)
            return (s2, bsel, above, crossed)
        init = (zeros, zeros, zeros, zeros)
        _, bsel, above, _ = lax.fori_loop(0, 256, sbody, init)
        return bsel, krem - above

    for r in range(_SC_ROUNDS):
        base = (wid * _SC_ROUNDS + r) * _L
        pltpu.sync_copy(x_hbm.at[:, pl.ds(base, _L)], chunk)

        @plsc.parallel_loop(0, _B, unroll=16)
        def _(i):
            iv = plsc.bitcast(chunk[i, :], jnp.int32)
            zv = iv ^ (lax.shift_right_arithmetic(iv, 31) | sign)
            chunk[i, :] = plsc.bitcast(zv, jnp.float32)
            b0 = lax.shift_right_logical(zv, 24)
            plsc.addupdate_scatter(hist, [b0, lanes], ones)

        krem = jnp.full((_L,), _K, jnp.int32)
        prefix, krem = scan_hist(krem)

        for sm, sb in ((24, 16), (16, 8), (8, 0)):
            @plsc.parallel_loop(0, _B, unroll=16)
            def _(i, sm=sm, sb=sb, prefix=prefix):
                zv = plsc.bitcast(chunk[i, :], jnp.int32)
                match = lax.shift_right_logical(zv, sm) == prefix
                bj = lax.shift_right_logical(zv, sb) & 0xFF
                plsc.addupdate_scatter(hist, [bj, lanes], ones, mask=match)
            bsel, krem = scan_hist(krem)
            prefix = lax.shift_left(prefix, 8) | bsel

        ty = prefix ^ sign

        @plsc.parallel_loop(0, _B, unroll=16)
        def _(i, ty=ty):
            zv = plsc.bitcast(chunk[i, :], jnp.int32)
            yv = zv ^ sign
            keep = yv >= ty
            flip = lax.shift_right_arithmetic(yv, 31) & m7f
            xv = plsc.bitcast(yv ^ flip, jnp.float32)
            chunk[i, :] = jnp.where(keep, xv, jnp.float32(0.0))

        pltpu.sync_copy(chunk, o_hbm.at[:, pl.ds(base, _L)])


def _sc_kernel(x_left):
    mesh = plsc.VectorSubcoreMesh(core_axis_name="c", subcore_axis_name="s")
    kern = pl.kernel(
        _sc_body,
        out_type=jax.ShapeDtypeStruct((_B, _N_SC), jnp.float32),
        mesh=mesh,
        scratch_types=[
            pltpu.VMEM((_B, _L), jnp.float32),
            pltpu.VMEM((256, _L), jnp.int32),
        ],
        compiler_params=pltpu.CompilerParams(use_tc_tiling_on_sc=False,
                                             needs_layout_passes=False),
    )
    return kern(x_left)


def _count_ge(key, cand, strict=False):
    def rec(lo, hi):
        if hi - lo == 16:
            m = (key[lo:hi] > cand) if strict else (key[lo:hi] >= cand)
            return jnp.where(m, jnp.int16(1), jnp.int16(0))
        mid = (lo + hi) // 2
        return rec(lo, mid) + rec(mid, hi)
    s = rec(0, key.shape[0])
    return jnp.sum(s.astype(jnp.int32), axis=0, keepdims=True)


def _tc_block_kernel(x_ref, o_ref):
    x = x_ref[...]
    i = lax.bitcast_convert_type(x, jnp.int32)
    flip = lax.shift_right_arithmetic(i, 31) & jnp.int32(0x7FFFFFFF)
    y = i ^ flip
    z = lax.bitcast_convert_type(y, jnp.uint32) ^ jnp.uint32(0x80000000)

    hi32 = (z >> jnp.uint32(16)).astype(jnp.int32)
    hs = (hi32 - 32768).astype(jnp.int16)
    p = jnp.zeros((1, _W), jnp.int32)
    for bit in range(15, -1, -1):
        cand = p | (1 << bit)
        ok = _count_ge(hs, (cand - 32768).astype(jnp.int16)) >= _K
        p = jnp.where(ok, cand, p)

    ps = (p - 32768).astype(jnp.int16)
    budget = _K - _count_ge(hs, ps, strict=True)
    lo32 = (z & jnp.uint32(0xFFFF)).astype(jnp.int32)
    los = jnp.where(hs == ps, (lo32 - 32768).astype(jnp.int16),
                    jnp.int16(-32768))
    q = jnp.zeros((1, _W), jnp.int32)
    for bit in range(15, -1, -1):
        cand = q | (1 << bit)
        ok = _count_ge(los, (cand - 32768).astype(jnp.int16)) >= budget
        q = jnp.where(ok, cand, q)

    thr = lax.bitcast_convert_type((p << 16) | q, jnp.uint32)
    keep = z >= thr
    o_ref[...] = jnp.where(keep, x, 0.0)


def _tc_kernel(x):
    grid = (_N_TC // _W,)
    off = _N_SC // _W
    return pl.pallas_call(
        _tc_block_kernel,
        grid=grid,
        in_specs=[pl.BlockSpec((_B, _W), lambda j: (0, j + off))],
        out_specs=pl.BlockSpec((_B, _W), lambda j: (0, j)),
        out_shape=jax.ShapeDtypeStruct((_B, _N_TC), jnp.float32),
    )(x)


def kernel(x):
    out_sc = _sc_kernel(x[:, :_N_SC])
    out_tc = _tc_kernel(x)
    return jnp.concatenate([out_sc, out_tc], axis=1)


# speedup vs baseline: 1.0059x; 1.0059x over previous
import jax
import jax.numpy as jnp
from jax import lax
from jax.experimental import pallas as pl
from jax.experimental.pallas import tpu as pltpu
from jax.experimental.pallas import tpu_sc as plsc

_B = 4096
_N = 4096
_K = 2048
_L = 16
_NW = 32
_N_SC = 1024
_SC_ROUNDS = _N_SC // (_NW * _L)
_W = 256
_N_TC = _N - _N_SC


def _sc_body(x_hbm, o_hbm, chunk, hist):
    wid = lax.axis_index("c") * 16 + lax.axis_index("s")
    lanes = lax.iota(jnp.int32, 16)
    ones = jnp.ones((_L,), jnp.int32)
    zeros = jnp.zeros((_L,), jnp.int32)
    sign = jnp.full((_L,), -0x80000000, jnp.int32)
    m7f = jnp.full((_L,), 0x7FFFFFFF, jnp.int32)

    @plsc.parallel_loop(0, 256, unroll=8)
    def _(b):
        hist[b, :] = zeros

    def scan_hist(krem):
        def sbody(j, car):
            s, bsel, above, crossed = car
            b = 255 - j
            hv = hist[b, :]
            hist[b, :] = zeros
            s2 = s + hv
            cross_now = (s2 >= krem) & (crossed == 0)
            bsel = jnp.where(cross_now, b, bsel)
            above = jnp.where(cross_now, s, above)
            crossed = jnp.where(cross_now, ones, crossed)
            return (s2, bsel, above, crossed)
        init = (zeros, zeros, zeros, zeros)
        _, bsel, above, _ = lax.fori_loop(0, 256, sbody, init)
        return bsel, krem - above

    for r in range(_SC_ROUNDS):
        base = (wid * _SC_ROUNDS + r) * _L
        pltpu.sync_copy(x_hbm.at[:, pl.ds(base, _L)], chunk)

        @plsc.parallel_loop(0, _B, unroll=16)
        def _(i):
            iv = plsc.bitcast(chunk[i, :], jnp.int32)
            zv = iv ^ (lax.shift_right_arithmetic(iv, 31) | sign)
            chunk[i, :] = plsc.bitcast(zv, jnp.float32)
            b0 = lax.shift_right_logical(zv, 24)
            plsc.addupdate_scatter(hist, [b0, lanes], ones)

        krem = jnp.full((_L,), _K, jnp.int32)
        prefix, krem = scan_hist(krem)

        for sm, sb in ((24, 16), (16, 8), (8, 0)):
            @plsc.parallel_loop(0, _B, unroll=16)
            def _(i, sm=sm, sb=sb, prefix=prefix):
                zv = plsc.bitcast(chunk[i, :], jnp.int32)
                match = lax.shift_right_logical(zv, sm) == prefix
                bj = lax.shift_right_logical(zv, sb) & 0xFF
                plsc.addupdate_scatter(hist, [bj, lanes], ones, mask=match)
            bsel, krem = scan_hist(krem)
            prefix = lax.shift_left(prefix, 8) | bsel

        ty = prefix ^ sign

        @plsc.parallel_loop(0, _B, unroll=16)
        def _(i, ty=ty):
            zv = plsc.bitcast(chunk[i, :], jnp.int32)
            yv = zv ^ sign
            keep = yv >= ty
            flip = lax.shift_right_arithmetic(yv, 31) & m7f
            xv = plsc.bitcast(yv ^ flip, jnp.float32)
            chunk[i, :] = jnp.where(keep, xv, jnp.float32(0.0))

        pltpu.sync_copy(chunk, o_hbm.at[:, pl.ds(base, _L)])


def _sc_kernel(x_left):
    mesh = plsc.VectorSubcoreMesh(core_axis_name="c", subcore_axis_name="s")
    kern = pl.kernel(
        _sc_body,
        out_type=jax.ShapeDtypeStruct((_B, _N_SC), jnp.float32),
        mesh=mesh,
        scratch_types=[
            pltpu.VMEM((_B, _L), jnp.float32),
            pltpu.VMEM((256, _L), jnp.int32),
        ],
        compiler_params=pltpu.CompilerParams(use_tc_tiling_on_sc=False,
                                             needs_layout_passes=False),
    )
    return kern(x_left)


def _count_ge(key, cand, strict=False):
    def rec(lo, hi):
        if hi - lo == 16:
            m = (key[lo:hi] > cand) if strict else (key[lo:hi] >= cand)
            return jnp.where(m, jnp.int16(1), jnp.int16(0))
        mid = (lo + hi) // 2
        return rec(lo, mid) + rec(mid, hi)
    s = rec(0, key.shape[0])
    return jnp.sum(s.astype(jnp.int32), axis=0, keepdims=True)


def _tc_block_kernel(x_ref, o_ref):
    x = x_ref[...]
    i = lax.bitcast_convert_type(x, jnp.int32)
    flip = lax.shift_right_arithmetic(i, 31) & jnp.int32(0x7FFFFFFF)
    y = i ^ flip
    z = lax.bitcast_convert_type(y, jnp.uint32) ^ jnp.uint32(0x80000000)

    hi32 = (z >> jnp.uint32(16)).astype(jnp.int32)
    hs = (hi32 - 32768).astype(jnp.int16)
    p = jnp.zeros((1, _W), jnp.int32)
    for bit in range(15, -1, -1):
        cand = p | (1 << bit)
        ok = _count_ge(hs, (cand - 32768).astype(jnp.int16)) >= _K
        p = jnp.where(ok, cand, p)

    ps = (p - 32768).astype(jnp.int16)
    budget = _K - _count_ge(hs, ps, strict=True)
    lo32 = (z & jnp.uint32(0xFFFF)).astype(jnp.int32)
    los = jnp.where(hs == ps, (lo32 - 32768).astype(jnp.int16),
                    jnp.int16(-32768))
    q = jnp.zeros((1, _W), jnp.int32)
    for bit in range(15, -1, -1):
        cand = q | (1 << bit)
        ok = _count_ge(los, (cand - 32768).astype(jnp.int16)) >= budget
        q = jnp.where(ok, cand, q)

    thr = lax.bitcast_convert_type((p << 16) | q, jnp.uint32)
    keep = z >= thr
    o_ref[...] = jnp.where(keep, x, 0.0)


def _tc_kernel(x):
    grid = (_N_TC // _W,)
    off = _N_SC // _W
    return pl.pallas_call(
        _tc_block_kernel,
        grid=grid,
        in_specs=[pl.BlockSpec((_B, _W), lambda j: (0, j + off))],
        out_specs=pl.BlockSpec((_B, _W), lambda j: (0, j)),
        out_shape=jax.ShapeDtypeStruct((_B, _N_TC), jnp.float32),
    )(x)


def kernel(x):
    out_tc = _tc_kernel(x)
    out_sc = _sc_kernel(x[:, :_N_SC])
    return jnp.concatenate([out_sc, out_tc], axis=1)
